# baseline (speedup 1.0000x reference)
import functools

import jax
import jax.numpy as jnp
from jax import lax
from jax.experimental import pallas as pl
from jax.experimental.pallas import tpu as pltpu
from jax.experimental.pallas import tpu_sc as plsc

L = 16
NBINS = 256
NWORKERS = 32
CHUNK = 16384


def _make_equalize(n_planes: int, h: int, w: int):
    n_pix = h * w
    assert n_planes % NWORKERS == 0
    assert n_pix % CHUNK == 0 and CHUNK % w == 0 and w % L == 0
    rows = CHUNK // w
    planes_per_w = n_planes // NWORKERS
    nchunks = n_pix // CHUNK
    wshift = w.bit_length() - 1
    assert w == 1 << wshift

    mesh = plsc.VectorSubcoreMesh(core_axis_name="c", subcore_axis_name="s")

    @functools.partial(
        pl.kernel,
        out_type=jax.ShapeDtypeStruct((n_planes, h, w), jnp.float32),
        mesh=mesh,
        compiler_params=pltpu.CompilerParams(
            needs_layout_passes=False, use_tc_tiling_on_sc=True),
        scratch_types=[
            pltpu.VMEM((rows, w), jnp.float32),
            pltpu.VMEM((rows, w), jnp.float32),
            pltpu.VMEM((rows, w), jnp.float32),
            pltpu.VMEM((rows, w), jnp.float32),
            pltpu.VMEM((L * NBINS,), jnp.int32),
            pltpu.VMEM((NBINS,), jnp.float32),
            pltpu.SemaphoreType.DMA((2,)),
            pltpu.SemaphoreType.DMA((2,)),
        ],
    )
    def eq_kernel(img_hbm, out_hbm, in_a, in_b, out_a, out_b, hist, lut,
                  isems, osems):
        wid = lax.axis_index("s") * 2 + lax.axis_index("c")
        iota16 = lax.iota(jnp.int32, L)
        lane_base = iota16 * NBINS
        ones = jnp.ones((L,), jnp.int32)
        ibufs = [in_a, in_b]
        obufs = [out_a, out_b]

        def per_plane(j, _):
            plane = wid * planes_per_w + j

            def in_cp(c):
                return pltpu.make_async_copy(
                    img_hbm.at[plane, pl.ds(c * rows, rows), :],
                    ibufs[c % 2], isems.at[c % 2])

            def out_cp(c):
                return pltpu.make_async_copy(
                    obufs[c % 2],
                    out_hbm.at[plane, pl.ds(c * rows, rows), :],
                    osems.at[c % 2])

            in_cp(0).start()

            def zero_body(t, c):
                hist[pl.ds(t * L, L)] = jnp.zeros((L,), jnp.int32)
                return c
            lax.fori_loop(0, (L * NBINS) // L, zero_body, 0)

            for c in range(nchunks):
                if c + 1 < nchunks:
                    in_cp(c + 1).start()
                in_cp(c).wait()
                buf = ibufs[c % 2]

                @plsc.parallel_loop(0, CHUNK, step=L, unroll=8)
                def hist_body(i):
                    v = buf[i >> wshift, pl.ds(i & (w - 1), L)]
                    idx = v.astype(jnp.int32)
                    plsc.addupdate_scatter(hist, [idx + lane_base], ones)

            in_cp(0).start()
            in_cp(1).start()

            def merge_body(t, carry):
                csum, mvec = carry
                acc = hist[pl.ds(t * L, L)]
                for ln in range(1, L):
                    acc = acc + hist[pl.ds(ln * NBINS + t * L, L)]
                inc = plsc.cumsum(acc) + csum
                hist[pl.ds(t * L, L)] = inc - acc
                mvec = jnp.maximum(mvec, jnp.where(inc < n_pix, inc, 0))
                return (csum + jnp.sum(acc), mvec)

            _, mvec = lax.fori_loop(
                0, NBINS // L, merge_body,
                (jnp.int32(0), jnp.zeros((L,), jnp.int32)))
            m = jnp.max(mvec)

            q = (m.astype(jnp.float32) * jnp.float32(1.0 / 255.0)) \
                .astype(jnp.int32)
            q = q - jnp.where(q * 255 > m, 1, 0)
            q = q + jnp.where((q + 1) * 255 <= m, 1, 0)
            step = q
            s2 = step >> 1
            ms = jnp.maximum(step, 1)
            msf = ms.astype(jnp.float32)
            seed = lax.bitcast_convert_type(
                jnp.int32(0x7EF477D5)
                - lax.bitcast_convert_type(msf, jnp.int32),
                jnp.float32)
            recip = seed
            for _ in range(3):
                recip = recip * (jnp.float32(2.0) - msf * recip)
            nz01 = jnp.where(step == 0, 0, 1)

            def lut_body(t, c):
                ce = hist[pl.ds(t * L, L)]
                x = ce + s2
                qi = (x.astype(jnp.float32) * recip).astype(jnp.int32)
                r = qi * ms
                qi = qi - jnp.where(r > x, 1, 0)
                qi = qi + jnp.where(r + ms <= x, 1, 0)
                lutv = jnp.clip(qi, 0, 255)
                vbase = t * L + iota16
                lutv = vbase + (lutv - vbase) * nz01
                lut[pl.ds(t * L, L)] = lutv.astype(jnp.float32)
                return c
            lax.fori_loop(0, NBINS // L, lut_body, 0)

            for c in range(nchunks):
                buf = ibufs[c % 2]
                obuf = obufs[c % 2]
                in_cp(c).wait()
                if c >= 2:
                    out_cp(c - 2).wait()

                @plsc.parallel_loop(0, CHUNK, step=L, unroll=8)
                def gather_body(i):
                    r = i >> wshift
                    cc = i & (w - 1)
                    v = buf[r, pl.ds(cc, L)]
                    idx = v.astype(jnp.int32)
                    obuf[r, pl.ds(cc, L)] = plsc.load_gather(lut, [idx])

                out_cp(c).start()
                if c + 2 < nchunks:
                    in_cp(c + 2).start()

            out_cp(nchunks - 2).wait()
            out_cp(nchunks - 1).wait()
            return 0

        lax.fori_loop(0, planes_per_w, per_plane, 0)

    return eq_kernel


def kernel(img, target):
    B, C, H, W = img.shape
    n_planes = B * C
    flat = img.reshape(n_planes, H, W)
    out = _make_equalize(n_planes, H, W)(flat)
    return out.reshape(B, C, H, W), target


# speedup vs baseline: 1.2561x; 1.2561x over previous
import functools

import jax
import jax.numpy as jnp
from jax import lax
from jax.experimental import pallas as pl
from jax.experimental.pallas import tpu as pltpu
from jax.experimental.pallas import tpu_sc as plsc

L = 16
NBINS = 256
NWORKERS = 32
CHUNK = 8192
PK = plsc.PackFormat.INTERLEAVED


def _make_equalize(n_planes: int, h: int, w: int):
    n_pix = h * w
    assert n_planes % NWORKERS == 0
    assert n_pix % CHUNK == 0 and CHUNK % w == 0 and w % (4 * L) == 0
    rows = CHUNK // w
    planes_per_w = n_planes // NWORKERS
    nchunks = n_pix // CHUNK
    wshift = w.bit_length() - 1
    assert w == 1 << wshift

    mesh = plsc.VectorSubcoreMesh(core_axis_name="c", subcore_axis_name="s")

    @functools.partial(
        pl.kernel,
        out_type=jax.ShapeDtypeStruct((n_planes, h, w), jnp.float32),
        mesh=mesh,
        compiler_params=pltpu.CompilerParams(
            needs_layout_passes=False, use_tc_tiling_on_sc=True),
        scratch_types=[
            pltpu.VMEM((rows, w), jnp.float32),
            pltpu.VMEM((rows, w), jnp.float32),
            pltpu.VMEM((rows, w), jnp.float32),
            pltpu.VMEM((rows, w), jnp.float32),
            pltpu.VMEM((n_pix // 4,), jnp.int32),
            pltpu.VMEM((L * NBINS,), jnp.int32),
            pltpu.VMEM((NBINS,), jnp.float32),
            pltpu.SemaphoreType.DMA((2,)),
            pltpu.SemaphoreType.DMA((2,)),
        ],
    )
    def eq_kernel(img_hbm, out_hbm, in_a, in_b, out_a, out_b, packed,
                  hist, lut, isems, osems):
        wid = lax.axis_index("s") * 2 + lax.axis_index("c")
        iota16 = lax.iota(jnp.int32, L)
        lane_base = iota16 * NBINS
        ones = jnp.ones((L,), jnp.int32)
        ibufs = [in_a, in_b]
        obufs = [out_a, out_b]

        def make_in_cp(plane):
            def in_cp(c):
                return pltpu.make_async_copy(
                    img_hbm.at[plane, pl.ds(c * rows, rows), :],
                    ibufs[c % 2], isems.at[c % 2])
            return in_cp

        first_plane = wid * planes_per_w
        make_in_cp(first_plane)(0).start()
        make_in_cp(first_plane)(1).start()

        def per_plane(j, _):
            plane = wid * planes_per_w + j
            in_cp = make_in_cp(plane)
            in_cp_next = make_in_cp(plane + 1)

            def out_cp(c):
                return pltpu.make_async_copy(
                    obufs[c % 2],
                    out_hbm.at[plane, pl.ds(c * rows, rows), :],
                    osems.at[c % 2])

            def zero_body(t, c):
                hist[pl.ds(t * L, L)] = jnp.zeros((L,), jnp.int32)
                return c
            lax.fori_loop(0, (L * NBINS) // L, zero_body, 0)

            for c in range(nchunks):
                in_cp(c).wait()
                buf = ibufs[c % 2]
                pbase = (c * CHUNK) >> 2

                @plsc.parallel_loop(0, CHUNK, step=4 * L, unroll=2)
                def hist_body(i):
                    r = i >> wshift
                    cb = i & (w - 1)
                    idxs = []
                    for k in range(4):
                        v = buf[r, pl.ds(cb + k * L, L)]
                        idx = jnp.clip(v, 0.0, 255.0).astype(jnp.int32)
                        plsc.addupdate_scatter(
                            hist, [idx + lane_base], ones)
                        idxs.append(idx)
                    p01 = plsc.pack(idxs[0], idxs[1], format=PK,
                                    preferred_element_type=jnp.int16)
                    p23 = plsc.pack(idxs[2], idxs[3], format=PK,
                                    preferred_element_type=jnp.int16)
                    p8 = plsc.pack(p01, p23, format=PK,
                                   preferred_element_type=jnp.int8)
                    packed[pl.ds(pbase + (i >> 2), L)] = \
                        plsc.bitcast(p8, jnp.int32)

                if c + 2 < nchunks:
                    in_cp(c + 2).start()
                else:
                    @pl.when(j < planes_per_w - 1)
                    def _prefetch():
                        in_cp_next(c + 2 - nchunks).start()

            def merge_body(t, carry):
                csum, mvec = carry
                acc = hist[pl.ds(t * L, L)]
                for ln in range(1, L):
                    acc = acc + hist[pl.ds(ln * NBINS + t * L, L)]
                inc = plsc.cumsum(acc) + csum
                hist[pl.ds(t * L, L)] = inc - acc
                mvec = jnp.maximum(mvec, jnp.where(inc < n_pix, inc, 0))
                return (csum + jnp.sum(acc), mvec)

            _, mvec = lax.fori_loop(
                0, NBINS // L, merge_body,
                (jnp.int32(0), jnp.zeros((L,), jnp.int32)))
            m = jnp.max(mvec)

            q = (m.astype(jnp.float32) * jnp.float32(1.0 / 255.0)) \
                .astype(jnp.int32)
            q = q - jnp.where(q * 255 > m, 1, 0)
            q = q + jnp.where((q + 1) * 255 <= m, 1, 0)
            step = q
            s2 = step >> 1
            ms = jnp.maximum(step, 1)
            msf = ms.astype(jnp.float32)
            seed = lax.bitcast_convert_type(
                jnp.int32(0x7EF477D5)
                - lax.bitcast_convert_type(msf, jnp.int32),
                jnp.float32)
            recip = seed
            for _ in range(3):
                recip = recip * (jnp.float32(2.0) - msf * recip)
            nz01 = jnp.where(step == 0, 0, 1)

            def lut_body(t, c):
                ce = hist[pl.ds(t * L, L)]
                x = ce + s2
                qi = (x.astype(jnp.float32) * recip).astype(jnp.int32)
                r = qi * ms
                qi = qi - jnp.where(r > x, 1, 0)
                qi = qi + jnp.where(r + ms <= x, 1, 0)
                lutv = jnp.clip(qi, 0, 255)
                vbase = t * L + iota16
                lutv = vbase + (lutv - vbase) * nz01
                lut[pl.ds(t * L, L)] = lutv.astype(jnp.float32)
                return c
            lax.fori_loop(0, NBINS // L, lut_body, 0)

            for c in range(nchunks):
                obuf = obufs[c % 2]
                pbase = (c * CHUNK) >> 2
                if c >= 2:
                    out_cp(c - 2).wait()

                @plsc.parallel_loop(0, CHUNK, step=4 * L, unroll=2)
                def gather_body(i):
                    r = i >> wshift
                    cb = i & (w - 1)
                    pw = packed[pl.ds(pbase + (i >> 2), L)]
                    p01, p23 = plsc.unpack(
                        plsc.bitcast(pw, jnp.int8), format=PK,
                        preferred_element_type=jnp.int16)
                    i0, i1 = plsc.unpack(
                        p01, format=PK, preferred_element_type=jnp.int32)
                    i2, i3 = plsc.unpack(
                        p23, format=PK, preferred_element_type=jnp.int32)
                    for k, idx in enumerate((i0, i1, i2, i3)):
                        obuf[r, pl.ds(cb + k * L, L)] = \
                            plsc.load_gather(lut, [idx & 255])

                out_cp(c).start()

            out_cp(nchunks - 2).wait()
            out_cp(nchunks - 1).wait()
            return 0

        lax.fori_loop(0, planes_per_w, per_plane, 0)

    return eq_kernel


def kernel(img, target):
    B, C, H, W = img.shape
    n_planes = B * C
    flat = img.reshape(n_planes, H, W)
    out = _make_equalize(n_planes, H, W)(flat)
    return out.reshape(B, C, H, W), target


# speedup vs baseline: 1.2903x; 1.0273x over previous
import functools

import jax
import jax.numpy as jnp
from jax import lax
from jax.experimental import pallas as pl
from jax.experimental.pallas import tpu as pltpu
from jax.experimental.pallas import tpu_sc as plsc

L = 16
NBINS = 256
NWORKERS = 32
CHUNK = 8192
PK = plsc.PackFormat.INTERLEAVED


def _make_equalize(n_planes: int, h: int, w: int):
    n_pix = h * w
    assert n_planes % NWORKERS == 0
    assert n_pix % CHUNK == 0 and CHUNK % w == 0 and w % (4 * L) == 0
    rows = CHUNK // w
    planes_per_w = n_planes // NWORKERS
    nchunks = n_pix // CHUNK
    wshift = w.bit_length() - 1
    assert w == 1 << wshift

    mesh = plsc.VectorSubcoreMesh(core_axis_name="c", subcore_axis_name="s")

    @functools.partial(
        pl.kernel,
        out_type=jax.ShapeDtypeStruct((n_planes, h, w), jnp.float32),
        mesh=mesh,
        compiler_params=pltpu.CompilerParams(
            needs_layout_passes=False, use_tc_tiling_on_sc=True),
        scratch_types=[
            pltpu.VMEM((rows, w), jnp.float32),
            pltpu.VMEM((rows, w), jnp.float32),
            pltpu.VMEM((rows, w), jnp.float32),
            pltpu.VMEM((rows, w), jnp.float32),
            pltpu.VMEM((n_pix // 4,), jnp.int32),
            pltpu.VMEM((L * NBINS,), jnp.int32),
            pltpu.VMEM((NBINS,), jnp.float32),
            pltpu.SemaphoreType.DMA((2,)),
            pltpu.SemaphoreType.DMA((2,)),
        ],
    )
    def eq_kernel(img_hbm, out_hbm, in_a, in_b, out_a, out_b, packed,
                  hist, lut, isems, osems):
        wid = lax.axis_index("s") * 2 + lax.axis_index("c")
        iota16 = lax.iota(jnp.int32, L)
        lane_base = iota16 * NBINS
        ones = jnp.ones((L,), jnp.int32)
        ibufs = [in_a, in_b]
        obufs = [out_a, out_b]

        def make_in_cp(plane):
            def in_cp(c):
                return pltpu.make_async_copy(
                    img_hbm.at[plane, pl.ds(c * rows, rows), :],
                    ibufs[c % 2], isems.at[c % 2])
            return in_cp

        first_plane = wid * planes_per_w
        make_in_cp(first_plane)(0).start()
        make_in_cp(first_plane)(1).start()

        def per_plane(j, _):
            plane = wid * planes_per_w + j
            in_cp = make_in_cp(plane)
            in_cp_next = make_in_cp(plane + 1)

            def out_cp(c):
                return pltpu.make_async_copy(
                    obufs[c % 2],
                    out_hbm.at[plane, pl.ds(c * rows, rows), :],
                    osems.at[c % 2])

            def zero_body(t, c):
                hist[pl.ds(t * L, L)] = jnp.zeros((L,), jnp.int32)
                return c
            lax.fori_loop(0, (L * NBINS) // L, zero_body, 0)

            for c in range(nchunks):
                in_cp(c).wait()
                buf = ibufs[c % 2]
                pbase = (c * CHUNK) >> 2

                @plsc.parallel_loop(0, CHUNK, step=4 * L, unroll=2)
                def hist_body(i):
                    r = i >> wshift
                    cb = i & (w - 1)
                    idxs = []
                    for k in range(4):
                        v = buf[r, pl.ds(cb + k * L, L)]
                        idx = v.astype(jnp.int32)
                        plsc.addupdate_scatter(
                            hist, [idx + lane_base], ones)
                        idxs.append(idx)
                    p01 = plsc.pack(idxs[0], idxs[1], format=PK,
                                    preferred_element_type=jnp.int16)
                    p23 = plsc.pack(idxs[2], idxs[3], format=PK,
                                    preferred_element_type=jnp.int16)
                    p8 = plsc.pack(p01, p23, format=PK,
                                   preferred_element_type=jnp.int8)
                    packed[pl.ds(pbase + (i >> 2), L)] = \
                        plsc.bitcast(p8, jnp.int32)

                if c + 2 < nchunks:
                    in_cp(c + 2).start()
                else:
                    @pl.when(j < planes_per_w - 1)
                    def _prefetch():
                        in_cp_next(c + 2 - nchunks).start()

            def merge_body(t, carry):
                csum, mvec = carry
                acc = hist[pl.ds(t * L, L)]
                for ln in range(1, L):
                    acc = acc + hist[pl.ds(ln * NBINS + t * L, L)]
                inc = plsc.cumsum(acc) + csum
                hist[pl.ds(t * L, L)] = inc - acc
                mvec = jnp.maximum(mvec, jnp.where(inc < n_pix, inc, 0))
                return (csum + jnp.sum(acc), mvec)

            _, mvec = lax.fori_loop(
                0, NBINS // L, merge_body,
                (jnp.int32(0), jnp.zeros((L,), jnp.int32)))
            m = jnp.max(mvec)

            q = (m.astype(jnp.float32) * jnp.float32(1.0 / 255.0)) \
                .astype(jnp.int32)
            q = q - jnp.where(q * 255 > m, 1, 0)
            q = q + jnp.where((q + 1) * 255 <= m, 1, 0)
            step = q
            s2 = step >> 1
            ms = jnp.maximum(step, 1)
            msf = ms.astype(jnp.float32)
            seed = lax.bitcast_convert_type(
                jnp.int32(0x7EF477D5)
                - lax.bitcast_convert_type(msf, jnp.int32),
                jnp.float32)
            recip = seed
            for _ in range(3):
                recip = recip * (jnp.float32(2.0) - msf * recip)
            nz01 = jnp.where(step == 0, 0, 1)

            def lut_body(t, c):
                ce = hist[pl.ds(t * L, L)]
                x = ce + s2
                qi = (x.astype(jnp.float32) * recip).astype(jnp.int32)
                r = qi * ms
                qi = qi - jnp.where(r > x, 1, 0)
                qi = qi + jnp.where(r + ms <= x, 1, 0)
                lutv = jnp.clip(qi, 0, 255)
                vbase = t * L + iota16
                lutv = vbase + (lutv - vbase) * nz01
                lut[pl.ds(t * L, L)] = lutv.astype(jnp.float32)
                return c
            lax.fori_loop(0, NBINS // L, lut_body, 0)

            for c in range(nchunks):
                obuf = obufs[c % 2]
                pbase = (c * CHUNK) >> 2
                if c >= 2:
                    out_cp(c - 2).wait()

                @plsc.parallel_loop(0, CHUNK, step=4 * L, unroll=2)
                def gather_body(i):
                    r = i >> wshift
                    cb = i & (w - 1)
                    pw = packed[pl.ds(pbase + (i >> 2), L)]
                    p01, p23 = plsc.unpack(
                        plsc.bitcast(pw, jnp.int8), format=PK,
                        preferred_element_type=jnp.int16)
                    i0, i1 = plsc.unpack(
                        p01, format=PK, preferred_element_type=jnp.int32)
                    i2, i3 = plsc.unpack(
                        p23, format=PK, preferred_element_type=jnp.int32)
                    for k, idx in enumerate((i0, i1, i2, i3)):
                        obuf[r, pl.ds(cb + k * L, L)] = \
                            plsc.load_gather(lut, [idx & 255])

                out_cp(c).start()

            out_cp(nchunks - 2).wait()
            out_cp(nchunks - 1).wait()
            return 0

        lax.fori_loop(0, planes_per_w, per_plane, 0)

    return eq_kernel


def kernel(img, target):
    B, C, H, W = img.shape
    n_planes = B * C
    flat = img.reshape(n_planes, H, W)
    out = _make_equalize(n_planes, H, W)(flat)
    return out.reshape(B, C, H, W), target


# speedup vs baseline: 1.3716x; 1.0630x over previous
import functools

import jax
import jax.numpy as jnp
from jax import lax
from jax.experimental import pallas as pl
from jax.experimental.pallas import tpu as pltpu
from jax.experimental.pallas import tpu_sc as plsc

L = 16
NBINS = 256
NWORKERS = 32
CHUNK_IN = 16384
CHUNK_OUT = 8192
PK = plsc.PackFormat.INTERLEAVED


def _make_equalize(n_planes: int, h: int, w: int):
    n_pix = h * w
    assert n_planes % NWORKERS == 0
    assert n_pix % CHUNK_IN == 0 and CHUNK_IN % w == 0 and w % (4 * L) == 0
    assert n_pix % CHUNK_OUT == 0 and CHUNK_OUT % w == 0
    rows_in = CHUNK_IN // w
    rows_out = CHUNK_OUT // w
    planes_per_w = n_planes // NWORKERS
    nchunks_in = n_pix // CHUNK_IN
    nchunks_out = n_pix // CHUNK_OUT
    wshift = w.bit_length() - 1
    assert w == 1 << wshift

    mesh = plsc.VectorSubcoreMesh(core_axis_name="c", subcore_axis_name="s")

    @functools.partial(
        pl.kernel,
        out_type=jax.ShapeDtypeStruct((n_planes, h, w), jnp.float32),
        mesh=mesh,
        compiler_params=pltpu.CompilerParams(
            needs_layout_passes=False, use_tc_tiling_on_sc=True),
        scratch_types=[
            pltpu.VMEM((rows_in, w), jnp.float32),
            pltpu.VMEM((rows_in, w), jnp.float32),
            pltpu.VMEM((rows_out, w), jnp.float32),
            pltpu.VMEM((rows_out, w), jnp.float32),
            pltpu.VMEM((n_pix // 4,), jnp.int32),
            pltpu.VMEM((L * NBINS,), jnp.int32),
            pltpu.VMEM((NBINS,), jnp.float32),
            pltpu.SemaphoreType.DMA((2,)),
            pltpu.SemaphoreType.DMA((2,)),
        ],
    )
    def eq_kernel(img_hbm, out_hbm, in_a, in_b, out_a, out_b, packed,
                  hist, lut, isems, osems):
        wid = lax.axis_index("s") * 2 + lax.axis_index("c")
        iota16 = lax.iota(jnp.int32, L)
        lane_base = iota16 * NBINS
        ones = jnp.ones((L,), jnp.int32)
        ibufs = [in_a, in_b]
        obufs = [out_a, out_b]

        def make_in_cp(plane):
            def in_cp(c):
                return pltpu.make_async_copy(
                    img_hbm.at[plane, pl.ds(c * rows_in, rows_in), :],
                    ibufs[c % 2], isems.at[c % 2])
            return in_cp

        first_plane = wid * planes_per_w
        make_in_cp(first_plane)(0).start()
        make_in_cp(first_plane)(1).start()

        def per_plane(j, _):
            plane = wid * planes_per_w + j
            in_cp = make_in_cp(plane)
            in_cp_next = make_in_cp(plane + 1)

            def out_cp(c):
                return pltpu.make_async_copy(
                    obufs[c % 2],
                    out_hbm.at[plane, pl.ds(c * rows_out, rows_out), :],
                    osems.at[c % 2])

            def zero_body(t, c):
                hist[pl.ds(t * L, L)] = jnp.zeros((L,), jnp.int32)
                return c
            lax.fori_loop(0, (L * NBINS) // L, zero_body, 0)

            for c in range(nchunks_in):
                in_cp(c).wait()
                buf = ibufs[c % 2]
                pbase = (c * CHUNK_IN) >> 2

                @plsc.parallel_loop(0, CHUNK_IN, step=4 * L, unroll=2)
                def hist_body(i):
                    r = i >> wshift
                    cb = i & (w - 1)
                    idxs = []
                    for k in range(4):
                        v = buf[r, pl.ds(cb + k * L, L)]
                        idx = v.astype(jnp.int32)
                        plsc.addupdate_scatter(
                            hist, [idx + lane_base], ones)
                        idxs.append(idx)
                    p01 = plsc.pack(idxs[0], idxs[1], format=PK,
                                    preferred_element_type=jnp.int16)
                    p23 = plsc.pack(idxs[2], idxs[3], format=PK,
                                    preferred_element_type=jnp.int16)
                    p8 = plsc.pack(p01, p23, format=PK,
                                   preferred_element_type=jnp.int8)
                    packed[pl.ds(pbase + (i >> 2), L)] = \
                        plsc.bitcast(p8, jnp.int32)

                if c + 2 < nchunks_in:
                    in_cp(c + 2).start()
                else:
                    @pl.when(j < planes_per_w - 1)
                    def _prefetch():
                        in_cp_next(c + 2 - nchunks_in).start()

            def merge_body(t, carry):
                csum, mvec = carry
                acc = hist[pl.ds(t * L, L)]
                for ln in range(1, L):
                    acc = acc + hist[pl.ds(ln * NBINS + t * L, L)]
                inc = plsc.cumsum(acc) + csum
                hist[pl.ds(t * L, L)] = inc - acc
                mvec = jnp.maximum(mvec, jnp.where(inc < n_pix, inc, 0))
                return (csum + jnp.sum(acc), mvec)

            _, mvec = lax.fori_loop(
                0, NBINS // L, merge_body,
                (jnp.int32(0), jnp.zeros((L,), jnp.int32)))
            m = jnp.max(mvec)

            q = (m.astype(jnp.float32) * jnp.float32(1.0 / 255.0)) \
                .astype(jnp.int32)
            q = q - jnp.where(q * 255 > m, 1, 0)
            q = q + jnp.where((q + 1) * 255 <= m, 1, 0)
            step = q
            s2 = step >> 1
            ms = jnp.maximum(step, 1)
            msf = ms.astype(jnp.float32)
            seed = lax.bitcast_convert_type(
                jnp.int32(0x7EF477D5)
                - lax.bitcast_convert_type(msf, jnp.int32),
                jnp.float32)
            recip = seed
            for _ in range(3):
                recip = recip * (jnp.float32(2.0) - msf * recip)
            nz01 = jnp.where(step == 0, 0, 1)

            def lut_body(t, c):
                ce = hist[pl.ds(t * L, L)]
                x = ce + s2
                qi = (x.astype(jnp.float32) * recip).astype(jnp.int32)
                r = qi * ms
                qi = qi - jnp.where(r > x, 1, 0)
                qi = qi + jnp.where(r + ms <= x, 1, 0)
                lutv = jnp.clip(qi, 0, 255)
                vbase = t * L + iota16
                lutv = vbase + (lutv - vbase) * nz01
                lut[pl.ds(t * L, L)] = lutv.astype(jnp.float32)
                return c
            lax.fori_loop(0, NBINS // L, lut_body, 0)

            for c in range(nchunks_out):
                obuf = obufs[c % 2]
                pbase = (c * CHUNK_OUT) >> 2
                if c >= 2:
                    out_cp(c - 2).wait()

                @plsc.parallel_loop(0, CHUNK_OUT, step=4 * L, unroll=2)
                def gather_body(i):
                    r = i >> wshift
                    cb = i & (w - 1)
                    pw = packed[pl.ds(pbase + (i >> 2), L)]
                    p01, p23 = plsc.unpack(
                        plsc.bitcast(pw, jnp.int8), format=PK,
                        preferred_element_type=jnp.int16)
                    i0, i1 = plsc.unpack(
                        p01, format=PK, preferred_element_type=jnp.int32)
                    i2, i3 = plsc.unpack(
                        p23, format=PK, preferred_element_type=jnp.int32)
                    for k, idx in enumerate((i0, i1, i2, i3)):
                        obuf[r, pl.ds(cb + k * L, L)] = \
                            plsc.load_gather(lut, [idx & 255])

                out_cp(c).start()

            out_cp(nchunks_out - 2).wait()
            out_cp(nchunks_out - 1).wait()
            return 0

        lax.fori_loop(0, planes_per_w, per_plane, 0)

    return eq_kernel


def kernel(img, target):
    B, C, H, W = img.shape
    n_planes = B * C
    flat = img.reshape(n_planes, H, W)
    out = _make_equalize(n_planes, H, W)(flat)
    return out.reshape(B, C, H, W), target
